# baseline (speedup 1.0000x reference)
import jax
import jax.numpy as jnp
import numpy as np
from jax.experimental import pallas as pl
from jax.experimental.pallas import tpu as pltpu

_NA = 3
_NC = 80
_C = _NC + 5
_CT = _NA * _C

_G = 52
_G2 = _G * _G
_B = 64
_NP = 104
_NBLK = _G2 // _NP


def _yolo_kernel(stride_ref, x_ref, o_ref):
    k = pl.program_id(0)
    v = x_ref[...].reshape(_NP * _B, _CT)
    c = jax.lax.broadcasted_iota(jnp.int32, v.shape, 1)
    cm = c % _C
    r = jax.lax.broadcasted_iota(jnp.int32, v.shape, 0)
    p = r // _B + k * _NP
    gx = (p % _G).astype(jnp.float32)
    gy = (p // _G).astype(jnp.float32)
    grid = jnp.where(cm == 0, gx, gy)
    sig = jax.nn.sigmoid(v)
    ex = jnp.exp(v)
    aw = jnp.where(c < _C, 10.0, jnp.where(c < 2 * _C, 16.0, 33.0))
    ah = jnp.where(c < _C, 13.0, jnp.where(c < 2 * _C, 30.0, 23.0))
    anc = jnp.where(cm == 2, aw, ah)
    stride = stride_ref[0, 0]
    val = jnp.where(cm < 2, (sig + grid) * stride, jnp.where(cm < 4, ex * anc, sig))
    o_ref[...] = val.T


def kernel(x, img_dim):
    B = x.shape[0]
    g = x.shape[2]
    g2 = g * g
    stride = (jnp.float32(img_dim) / jnp.float32(g)).reshape(1, 1)

    xt = jnp.transpose(x, (2, 3, 0, 1)).reshape(g2, B, _CT)

    y = pl.pallas_call(
        _yolo_kernel,
        grid=(_NBLK,),
        in_specs=[
            pl.BlockSpec(memory_space=pltpu.SMEM),
            pl.BlockSpec((_NP, B, _CT), lambda k: (k, 0, 0)),
        ],
        out_specs=pl.BlockSpec((_CT, _NP * B), lambda k: (0, k)),
        out_shape=jax.ShapeDtypeStruct((_CT, g2 * B), jnp.float32),
    )(stride, xt)

    y4 = y.reshape(_NA, _C, g2, B)
    return y4.transpose(3, 0, 2, 1).reshape(B, _NA * g2, _C)


# speedup vs baseline: 1.0211x; 1.0211x over previous
import jax
import jax.numpy as jnp
import numpy as np
from jax.experimental import pallas as pl

_NUM_ANCHORS = 3
_NUM_CLASSES = 80
_ANCHORS = np.array([[10.0, 13.0], [16.0, 30.0], [33.0, 23.0]], dtype=np.float32)
_C = _NUM_CLASSES + 5


def _yolo_block_kernel(x_ref, scale_ref, o_ref):
    g2 = x_ref.shape[2]
    g = int(round(g2 ** 0.5))
    blk = x_ref[0]
    sig = jax.nn.sigmoid(blk)
    ex = jnp.exp(blk)
    row = jax.lax.broadcasted_iota(jnp.int32, (_C, g2), 0)
    col = jax.lax.broadcasted_iota(jnp.int32, (_C, g2), 1)
    gx = (col % g).astype(jnp.float32)
    gy = (col // g).astype(jnp.float32)
    grid = jnp.where(row == 0, gx, gy)
    val = jnp.where(row < 2, sig + grid, jnp.where(row < 4, ex, sig))
    o_ref[0] = val.T * scale_ref[0]


def kernel(x, img_dim):
    B = x.shape[0]
    g = x.shape[2]
    g2 = g * g
    stride = jnp.float32(img_dim) / jnp.float32(g)
    nblk = B * _NUM_ANCHORS

    ones = jnp.ones((_NUM_ANCHORS, _C - 4), dtype=jnp.float32)
    st2 = jnp.broadcast_to(stride, (_NUM_ANCHORS, 2))
    scales = jnp.concatenate([st2, jnp.asarray(_ANCHORS)], axis=1)
    scales = jnp.concatenate([scales, ones], axis=1).reshape(_NUM_ANCHORS, 1, _C)

    xr = x.reshape(nblk, _C, g2)

    out = pl.pallas_call(
        _yolo_block_kernel,
        grid=(nblk,),
        in_specs=[
            pl.BlockSpec((1, _C, g2), lambda i: (i, 0, 0)),
            pl.BlockSpec((1, 1, _C), lambda i: (i % _NUM_ANCHORS, 0, 0)),
        ],
        out_specs=pl.BlockSpec((1, g2, _C), lambda i: (i, 0, 0)),
        out_shape=jax.ShapeDtypeStruct((nblk, g2, _C), jnp.float32),
    )(xr, scales)

    return out.reshape(B, _NUM_ANCHORS * g2, _C)


# speedup vs baseline: 1.2243x; 1.1990x over previous
import jax
import jax.numpy as jnp
import numpy as np
from jax.experimental import pallas as pl

_NUM_ANCHORS = 3
_NUM_CLASSES = 80
_ANCHORS = np.array([[10.0, 13.0], [16.0, 30.0], [33.0, 23.0]], dtype=np.float32)
_C = _NUM_CLASSES + 5
_CT = _NUM_ANCHORS * _C


def _yolo_block_kernel(x_ref, scale_ref, o_ref):
    g = x_ref.shape[2]
    g2 = g * g
    blk = x_ref[0].reshape(_CT, g2)
    sig = jax.nn.sigmoid(blk)
    ex = jnp.exp(blk)
    row = jax.lax.broadcasted_iota(jnp.int32, (_CT, g2), 0)
    rm = row % _C
    col = jax.lax.broadcasted_iota(jnp.int32, (_CT, g2), 1)
    gx = (col % g).astype(jnp.float32)
    gy = (col // g).astype(jnp.float32)
    grid = jnp.where(rm == 0, gx, gy)
    val = jnp.where(rm < 2, sig + grid, jnp.where(rm < 4, ex, sig))
    for a in range(_NUM_ANCHORS):
        t = val[a * _C : (a + 1) * _C].T
        o_ref[0, a * g2 : (a + 1) * g2, :] = t * scale_ref[a]


def kernel(x, img_dim):
    B = x.shape[0]
    g = x.shape[2]
    g2 = g * g
    stride = jnp.float32(img_dim) / jnp.float32(g)

    ones = jnp.ones((_NUM_ANCHORS, _C - 4), dtype=jnp.float32)
    st2 = jnp.broadcast_to(stride, (_NUM_ANCHORS, 2))
    scales = jnp.concatenate([st2, jnp.asarray(_ANCHORS)], axis=1)
    scales = jnp.concatenate([scales, ones], axis=1).reshape(_NUM_ANCHORS, 1, _C)

    out = pl.pallas_call(
        _yolo_block_kernel,
        grid=(B,),
        in_specs=[
            pl.BlockSpec((1, _CT, g, g), lambda b: (b, 0, 0, 0)),
            pl.BlockSpec((_NUM_ANCHORS, 1, _C), lambda b: (0, 0, 0)),
        ],
        out_specs=pl.BlockSpec((1, _NUM_ANCHORS * g2, _C), lambda b: (b, 0, 0)),
        out_shape=jax.ShapeDtypeStruct((B, _NUM_ANCHORS * g2, _C), jnp.float32),
    )(x, scales)

    return out


# speedup vs baseline: 1.2549x; 1.0250x over previous
import jax
import jax.numpy as jnp
import numpy as np
from jax.experimental import pallas as pl

_NUM_ANCHORS = 3
_NUM_CLASSES = 80
_ANCHORS = np.array([[10.0, 13.0], [16.0, 30.0], [33.0, 23.0]], dtype=np.float32)
_C = _NUM_CLASSES + 5
_CT = _NUM_ANCHORS * _C


def _yolo_block_kernel(x_ref, scale_ref, o_ref):
    nb = x_ref.shape[0]
    g = x_ref.shape[2]
    g2 = g * g
    blk = x_ref[...].reshape(nb * _CT, g2)
    sig = jax.nn.sigmoid(blk)
    ex = jnp.exp(blk)
    row = jax.lax.broadcasted_iota(jnp.int32, (nb * _CT, g2), 0)
    rm = row % _C
    col = jax.lax.broadcasted_iota(jnp.int32, (nb * _CT, g2), 1)
    gx = (col % g).astype(jnp.float32)
    gy = (col // g).astype(jnp.float32)
    grid = jnp.where(rm == 0, gx, gy)
    val = jnp.where(rm < 2, sig + grid, jnp.where(rm < 4, ex, sig))
    for b in range(nb):
        for a in range(_NUM_ANCHORS):
            t = val[b * _CT + a * _C : b * _CT + (a + 1) * _C].T
            o_ref[b, a * g2 : (a + 1) * g2, :] = t * scale_ref[a]


def kernel(x, img_dim):
    B = x.shape[0]
    g = x.shape[2]
    g2 = g * g
    stride = jnp.float32(img_dim) / jnp.float32(g)

    ones = jnp.ones((_NUM_ANCHORS, _C - 4), dtype=jnp.float32)
    st2 = jnp.broadcast_to(stride, (_NUM_ANCHORS, 2))
    scales = jnp.concatenate([st2, jnp.asarray(_ANCHORS)], axis=1)
    scales = jnp.concatenate([scales, ones], axis=1).reshape(_NUM_ANCHORS, 1, _C)

    nb = 2
    out = pl.pallas_call(
        _yolo_block_kernel,
        grid=(B // nb,),
        in_specs=[
            pl.BlockSpec((nb, _CT, g, g), lambda b: (b, 0, 0, 0)),
            pl.BlockSpec((_NUM_ANCHORS, 1, _C), lambda b: (0, 0, 0)),
        ],
        out_specs=pl.BlockSpec((nb, _NUM_ANCHORS * g2, _C), lambda b: (b, 0, 0)),
        out_shape=jax.ShapeDtypeStruct((B, _NUM_ANCHORS * g2, _C), jnp.float32),
    )(x, scales)

    return out
